# baseline (speedup 1.0000x reference)
import functools

import jax
import jax.numpy as jnp
from jax import lax
from jax.experimental import pallas as pl
from jax.experimental.pallas import tpu as pltpu
from jax.experimental.pallas import tpu_sc as plsc

_D = 768
_GRID = 32
_LANES = 16
_NCHUNK = _D // _LANES
_NW = 32
_GROUP = 16
_TMAX = 33
_HB = 32


def _dembed_body(xP_hbm, tok_hbm, col_hbm, row_hbm, ext_hbm, out_hbm,
                 idxall_v, oidx_v, cbuf_v, rowrow_v, pos_v, rows_v,
                 gsem, ssem, csem):
    NWK, TMAX, B = xP_hbm.shape
    N = NWK * (TMAX - 1) + 1
    wid = lax.axis_index("s") * 2 + lax.axis_index("c")
    trips = jnp.where(wid == 0, TMAX, TMAX - 1)
    U = 2 * trips

    def gather_desc(u):
        t, h, s = u // 2, lax.rem(u, 2), lax.rem(u, 4)
        return pltpu.make_async_copy(
            tok_hbm.at[idxall_v.at[t, pl.ds(_HB * h, _HB)]],
            rows_v.at[s], gsem.at[s])

    def scatter_descs(s):
        return [pltpu.make_async_copy(
                    rows_v.at[s, :, pl.ds(128 * c, 128)],
                    out_hbm.at[oidx_v.at[s, c]], ssem.at[s])
                for c in range(6)]

    def scatter_start(s):
        for d in scatter_descs(s):
            d.start()

    def scatter_wait(s):
        for d in scatter_descs(s):
            d.wait()

    def cidx_of(t):
        return jnp.where(wid == 0, t - 1, t)

    pltpu.sync_copy(xP_hbm.at[wid], idxall_v)
    gather_desc(0).start()
    gather_desc(1).start()
    pltpu.sync_copy(row_hbm.at[lax.rem(wid + _GRID - 1, _GRID)], rowrow_v)

    @pl.when(wid > 0)
    def _():
        pltpu.sync_copy(col_hbm.at[cidx_of(0)], cbuf_v.at[0])

    def body(u, carry):
        t, h, s = u // 2, lax.rem(u, 2), lax.rem(u, 4)
        n = wid + _NW * t
        cp = lax.rem(t, 2)

        @pl.when(u + 2 < U)
        def _():
            @pl.when(u >= 2)
            def _():
                scatter_wait(lax.rem(u + 2, 4))

            gather_desc(u + 2).start()

        gather_desc(u).wait()

        @pl.when(h == 0)
        def _():
            @pl.when(t + 1 < trips)
            def _():
                pltpu.async_copy(col_hbm.at[cidx_of(t + 1)],
                                 cbuf_v.at[1 - cp], csem.at[1 - cp])

            @pl.when(t >= 1)
            def _():
                pltpu.make_async_copy(col_hbm.at[0], cbuf_v.at[cp],
                                      csem.at[cp]).wait()

            @pl.when(n == 0)
            def _():
                pltpu.sync_copy(ext_hbm.at[0], pos_v)

            @pl.when(n > 0)
            def _():
                for j in range(_NCHUNK):
                    ds = pl.ds(_LANES * j, _LANES)
                    pos_v[ds] = cbuf_v[cp, ds] + rowrow_v[ds]

        io = lax.iota(jnp.int32, _LANES)
        hi48 = jnp.where(io >= 8, io + 40, io)
        for c in range(6):
            for k in range(_HB // _LANES):
                oidx_v[s, c, pl.ds(_LANES * k, _LANES)] = (
                    12288 * wid + 384 * lax.rem(t, _GRID) + 192 * h + 96 * k
                    + 8 * c) + hi48

        for g in range(_NCHUNK // _GROUP):
            base = g * _GROUP * _LANES
            pvs = tuple(pos_v[pl.ds(base + _LANES * j, _LANES)]
                        for j in range(_GROUP))

            @plsc.parallel_loop(0, _HB, carry=pvs)
            def rowbody(b, pv, base=base):
                for j in range(_GROUP):
                    ds = pl.ds(base + _LANES * j, _LANES)
                    rows_v[s, b, ds] = rows_v[s, b, ds] + pv[j]
                return pv

        scatter_start(s)
        return carry

    lax.fori_loop(0, U, body, 0)

    for d in range(4):
        scatter_wait(lax.rem(U - 4 + d, 4))


def kernel(x, tok_table, col_table, row_table, ext_table):
    B, N = x.shape
    xT = x.T
    xP = jnp.concatenate(
        [xT[: _NW * (_TMAX - 1)].reshape(_TMAX - 1, _NW, B).transpose(1, 0, 2),
         jnp.broadcast_to(xT[_NW * (_TMAX - 1):], (_NW, 1, B))], axis=1)

    mesh = plsc.VectorSubcoreMesh(core_axis_name="c", subcore_axis_name="s")
    run = functools.partial(
        pl.kernel,
        out_type=jax.ShapeDtypeStruct((B * N * 6, 128), jnp.float32),
        mesh=mesh,
        scratch_types=[
            pltpu.VMEM((_TMAX, B), jnp.int32),
            pltpu.VMEM((4, 6, _HB), jnp.int32),
            pltpu.VMEM((2, _D), jnp.float32),
            pltpu.VMEM((_D,), jnp.float32),
            pltpu.VMEM((_D,), jnp.float32),
            pltpu.VMEM((4, _HB, _D), jnp.float32),
            pltpu.SemaphoreType.DMA((4,)),
            pltpu.SemaphoreType.DMA((4,)),
            pltpu.SemaphoreType.DMA((2,)),
        ],
    )(_dembed_body)
    out = run(xP, tok_table, col_table, row_table, ext_table)
    out = out.reshape(N, B // 8, 6, 8, 128)
    out = out.transpose(1, 3, 0, 2, 4)
    return out.reshape(B, N, _D)


# speedup vs baseline: 1.0053x; 1.0053x over previous
import functools

import jax
import jax.numpy as jnp
from jax import lax
from jax.experimental import pallas as pl
from jax.experimental.pallas import tpu as pltpu
from jax.experimental.pallas import tpu_sc as plsc

_D = 768
_GRID = 32
_LANES = 16
_NCHUNK = _D // _LANES
_NW = 32
_GROUP = 16
_TMAX = 33
_HB = 32


def _dembed_body(xP_hbm, tok_hbm, col_hbm, row_hbm, ext_hbm, out_hbm,
                 idxall_v, oidx_v, cbuf_v, rowrow_v, pos_v, rows_v,
                 gsem, ssem, csem):
    NWK, TMAX, B = xP_hbm.shape
    N = NWK * (TMAX - 1) + 1
    wid = lax.axis_index("s") * 2 + lax.axis_index("c")
    trips = jnp.where(wid == 0, TMAX, TMAX - 1)
    U = 2 * trips

    def gather_desc(u):
        t, h, s = u // 2, lax.rem(u, 2), lax.rem(u, 4)
        return pltpu.make_async_copy(
            tok_hbm.at[idxall_v.at[t, pl.ds(_HB * h, _HB)]],
            rows_v.at[s], gsem.at[s])

    def scatter_descs(s):
        return [pltpu.make_async_copy(
                    rows_v.at[s, :, pl.ds(128 * c, 128)],
                    out_hbm.at[oidx_v.at[s, c]], ssem.at[s])
                for c in range(6)]

    def scatter_start(s):
        for d in scatter_descs(s):
            d.start()

    def scatter_wait(s):
        for d in scatter_descs(s):
            d.wait()

    def cidx_of(t):
        return jnp.where(wid == 0, t - 1, t)

    pltpu.sync_copy(xP_hbm.at[wid], idxall_v)
    gather_desc(0).start()
    gather_desc(1).start()
    pltpu.sync_copy(row_hbm.at[lax.rem(wid + _GRID - 1, _GRID)], rowrow_v)

    @pl.when(wid > 0)
    def _():
        pltpu.sync_copy(col_hbm.at[cidx_of(0)], cbuf_v.at[0])

    def body(u, carry):
        t, h, s = u // 2, lax.rem(u, 2), lax.rem(u, 4)
        n = wid + _NW * t
        cp = lax.rem(t, 2)

        @pl.when(u + 2 < U)
        def _():
            @pl.when(u >= 2)
            def _():
                scatter_wait(lax.rem(u + 2, 4))

            gather_desc(u + 2).start()

        gather_desc(u).wait()

        @pl.when(h == 0)
        def _():
            @pl.when(t + 1 < trips)
            def _():
                pltpu.async_copy(col_hbm.at[cidx_of(t + 1)],
                                 cbuf_v.at[1 - cp], csem.at[1 - cp])

            @pl.when(t >= 1)
            def _():
                pltpu.make_async_copy(col_hbm.at[0], cbuf_v.at[cp],
                                      csem.at[cp]).wait()

            @pl.when(n == 0)
            def _():
                pltpu.sync_copy(ext_hbm.at[0], pos_v)

            @pl.when(n > 0)
            def _():
                for j in range(_NCHUNK):
                    ds = pl.ds(_LANES * j, _LANES)
                    pos_v[ds] = cbuf_v[cp, ds] + rowrow_v[ds]

        io = lax.iota(jnp.int32, _LANES)
        hi48 = jnp.where(io >= 8, io + 40, io)
        for c in range(6):
            for k in range(_HB // _LANES):
                oidx_v[s, c, pl.ds(_LANES * k, _LANES)] = (
                    384 * n + 192 * h + 96 * k + 8 * c) + hi48

        for g in range(_NCHUNK // _GROUP):
            base = g * _GROUP * _LANES
            pvs = tuple(pos_v[pl.ds(base + _LANES * j, _LANES)]
                        for j in range(_GROUP))

            @plsc.parallel_loop(0, _HB, carry=pvs)
            def rowbody(b, pv, base=base):
                for j in range(_GROUP):
                    ds = pl.ds(base + _LANES * j, _LANES)
                    rows_v[s, b, ds] = rows_v[s, b, ds] + pv[j]
                return pv

        scatter_start(s)
        return carry

    lax.fori_loop(0, U, body, 0)

    for d in range(4):
        scatter_wait(lax.rem(U - 4 + d, 4))


def kernel(x, tok_table, col_table, row_table, ext_table):
    B, N = x.shape
    xT = x.T
    xP = jnp.concatenate(
        [xT[: _NW * (_TMAX - 1)].reshape(_TMAX - 1, _NW, B).transpose(1, 0, 2),
         jnp.broadcast_to(xT[_NW * (_TMAX - 1):], (_NW, 1, B))], axis=1)

    mesh = plsc.VectorSubcoreMesh(core_axis_name="c", subcore_axis_name="s")
    run = functools.partial(
        pl.kernel,
        out_type=jax.ShapeDtypeStruct((B * N * 6, 128), jnp.float32),
        mesh=mesh,
        scratch_types=[
            pltpu.VMEM((_TMAX, B), jnp.int32),
            pltpu.VMEM((4, 6, _HB), jnp.int32),
            pltpu.VMEM((2, _D), jnp.float32),
            pltpu.VMEM((_D,), jnp.float32),
            pltpu.VMEM((_D,), jnp.float32),
            pltpu.VMEM((4, _HB, _D), jnp.float32),
            pltpu.SemaphoreType.DMA((4,)),
            pltpu.SemaphoreType.DMA((4,)),
            pltpu.SemaphoreType.DMA((2,)),
        ],
    )(_dembed_body)
    out = run(xP, tok_table, col_table, row_table, ext_table)
    out = out.reshape(N, B // 8, 6, 8, 128)
    out = out.transpose(1, 3, 0, 2, 4)
    return out.reshape(B, N, _D)
